# initial kernel scaffold (unmeasured)
import functools

import jax
import jax.numpy as jnp
from jax import lax
from jax.experimental import pallas as pl
from jax.experimental.pallas import tpu as pltpu

N_DEV = 4
N_LAYERS = 3
TN = 512


def kernel(x, Win0, Wout0, Win1, Wout1, Win2, Wout2):
    B, D = x.shape
    H = Win0.shape[1]
    NT = H // TN

    def body(x_ref, win0, wout0, win1, wout1, win2, wout2, out_ref,
             xb, hb, acc, comm, wibuf, wobuf,
             wi_sems, wo_sems, send_sems, recv_sems):
        my = lax.axis_index("i")
        left = (my - 1 + N_DEV) % N_DEV
        right = (my + 1) % N_DEV

        bar = pltpu.get_barrier_semaphore()
        for nbr in (left, right):
            pl.semaphore_signal(bar, inc=1, device_id=(nbr,),
                                device_id_type=pl.DeviceIdType.MESH)
        pl.semaphore_wait(bar, 2)

        xb[...] = x_ref[...].astype(jnp.bfloat16)

        wins = [win0, win1, win2]
        wouts = [wout0, wout1, wout2]

        for L in range(N_LAYERS):
            win, wout = wins[L], wouts[L]

            def wi_copy(t, slot, win=win):
                return pltpu.make_async_copy(
                    win.at[:, pl.ds(t * TN, TN)], wibuf.at[slot],
                    wi_sems.at[slot])

            def wo_copy(t, slot, wout=wout):
                return pltpu.make_async_copy(
                    wout.at[pl.ds(t * TN, TN), :], wobuf.at[slot],
                    wo_sems.at[slot])

            wi_copy(0, 0).start()
            for t in range(NT):
                slot = t % 2
                if t + 1 < NT:
                    wi_copy(t + 1, (t + 1) % 2).start()
                wi_copy(t, slot).wait()
                hb[:, pl.ds(t * TN, TN)] = jnp.maximum(
                    jnp.dot(xb[...], wibuf[slot].astype(jnp.bfloat16),
                            preferred_element_type=jnp.float32),
                    0.0).astype(jnp.bfloat16)

            wo_copy(0, 0).start()
            for t in range(NT):
                slot = t % 2
                if t + 1 < NT:
                    wo_copy(t + 1, (t + 1) % 2).start()
                wo_copy(t, slot).wait()
                p = jnp.dot(hb[:, pl.ds(t * TN, TN)],
                            wobuf[slot].astype(jnp.bfloat16),
                            preferred_element_type=jnp.float32)
                if t == 0:
                    acc[...] = p
                else:
                    acc[...] = acc[...] + p

            comm[0, :, :] = acc[...].astype(jnp.bfloat16)
            for h in range(N_DEV - 1):
                rdma = pltpu.make_async_remote_copy(
                    src_ref=comm.at[h],
                    dst_ref=comm.at[h + 1],
                    send_sem=send_sems.at[L, h],
                    recv_sem=recv_sems.at[L, h],
                    device_id=(right,),
                    device_id_type=pl.DeviceIdType.MESH,
                )
                rdma.start()
                rdma.wait()

            total = acc[...]
            for h in range(1, N_DEV):
                total = total + comm[h, :, :].astype(jnp.float32)

            if L == N_LAYERS - 1:
                out_ref[...] = total
            else:
                xb[...] = total.astype(jnp.bfloat16)

        @functools.partial(pl.run_scoped, sem=pltpu.SemaphoreType.REGULAR)
        def _(sem):
            for nbr in (left, right):
                pl.semaphore_signal(sem, inc=1, device_id=(nbr,),
                                    device_id_type=pl.DeviceIdType.MESH)
            pl.semaphore_wait(sem, 2)

    return pl.pallas_call(
        body,
        out_shape=jax.ShapeDtypeStruct((B, D), jnp.float32),
        in_specs=[pl.BlockSpec(memory_space=pltpu.VMEM)]
        + [pl.BlockSpec(memory_space=pltpu.ANY)] * 6,
        out_specs=pl.BlockSpec(memory_space=pltpu.VMEM),
        scratch_shapes=[
            pltpu.VMEM((B, D), jnp.bfloat16),
            pltpu.VMEM((B, H), jnp.bfloat16),
            pltpu.VMEM((B, D), jnp.float32),
            pltpu.VMEM((N_DEV, B, D), jnp.bfloat16),
            pltpu.VMEM((2, D, TN), jnp.float32),
            pltpu.VMEM((2, TN, D), jnp.float32),
            pltpu.SemaphoreType.DMA((2,)),
            pltpu.SemaphoreType.DMA((2,)),
            pltpu.SemaphoreType.DMA((N_LAYERS, N_DEV - 1)),
            pltpu.SemaphoreType.DMA((N_LAYERS, N_DEV - 1)),
        ],
        compiler_params=pltpu.CompilerParams(collective_id=0),
    )(x, Win0, Wout0, Win1, Wout1, Win2, Wout2)


# baseline (device time: 118703 ns/iter reference)
import functools

import jax
import jax.numpy as jnp
from jax import lax
from jax.experimental import pallas as pl
from jax.experimental.pallas import tpu as pltpu

N_DEV = 4
N_LAYERS = 3
TN = 512


def kernel(x, Win0, Wout0, Win1, Wout1, Win2, Wout2):
    B, D = x.shape
    H = Win0.shape[1]
    NT = H // TN

    def body(x_ref, win0, wout0, win1, wout1, win2, wout2, out_ref,
             xb, hb, acc, comm, wibuf, wobuf,
             wi_sems, wo_sems, send_sems, recv_sems):
        my = lax.axis_index("i")
        left = (my - 1 + N_DEV) % N_DEV
        right = (my + 1) % N_DEV

        bar = pltpu.get_barrier_semaphore()
        for nbr in (left, right):
            pl.semaphore_signal(bar, inc=1, device_id=(nbr,),
                                device_id_type=pl.DeviceIdType.MESH)
        pl.semaphore_wait(bar, 2)

        xb[...] = x_ref[...].astype(jnp.bfloat16)

        wins = [win0, win1, win2]
        wouts = [wout0, wout1, wout2]

        for L in range(N_LAYERS):
            win, wout = wins[L], wouts[L]

            def wi_copy(t, slot, win=win):
                return pltpu.make_async_copy(
                    win.at[:, pl.ds(t * TN, TN)], wibuf.at[slot],
                    wi_sems.at[slot])

            def wo_copy(t, slot, wout=wout):
                return pltpu.make_async_copy(
                    wout.at[pl.ds(t * TN, TN), :], wobuf.at[slot],
                    wo_sems.at[slot])

            wi_copy(0, 0).start()
            for t in range(NT):
                slot = t % 2
                if t + 1 < NT:
                    wi_copy(t + 1, (t + 1) % 2).start()
                wi_copy(t, slot).wait()
                hb[:, pl.ds(t * TN, TN)] = jnp.maximum(
                    jnp.dot(xb[...], wibuf[slot].astype(jnp.bfloat16),
                            preferred_element_type=jnp.float32),
                    0.0).astype(jnp.bfloat16)

            wo_copy(0, 0).start()
            for t in range(NT):
                slot = t % 2
                if t + 1 < NT:
                    wo_copy(t + 1, (t + 1) % 2).start()
                wo_copy(t, slot).wait()
                p = jnp.dot(hb[:, pl.ds(t * TN, TN)],
                            wobuf[slot].astype(jnp.bfloat16),
                            preferred_element_type=jnp.float32)
                if t == 0:
                    acc[...] = p
                else:
                    acc[...] = acc[...] + p

            comm[0, :, :] = acc[...].astype(jnp.bfloat16)
            for h in range(N_DEV - 1):
                rdma = pltpu.make_async_remote_copy(
                    src_ref=comm.at[h],
                    dst_ref=comm.at[h + 1],
                    send_sem=send_sems.at[L, h],
                    recv_sem=recv_sems.at[L, h],
                    device_id=(right,),
                    device_id_type=pl.DeviceIdType.MESH,
                )
                rdma.start()
                rdma.wait()

            total = acc[...]
            for h in range(1, N_DEV):
                total = total + comm[h, :, :].astype(jnp.float32)

            if L == N_LAYERS - 1:
                out_ref[...] = total
            else:
                xb[...] = total.astype(jnp.bfloat16)

        @functools.partial(pl.run_scoped, sem=pltpu.SemaphoreType.REGULAR)
        def _(sem):
            for nbr in (left, right):
                pl.semaphore_signal(sem, inc=1, device_id=(nbr,),
                                    device_id_type=pl.DeviceIdType.MESH)
            pl.semaphore_wait(sem, 2)

    return pl.pallas_call(
        body,
        out_shape=jax.ShapeDtypeStruct((B, D), jnp.float32),
        in_specs=[pl.BlockSpec(memory_space=pltpu.VMEM)]
        + [pl.BlockSpec(memory_space=pl.ANY)] * 6,
        out_specs=pl.BlockSpec(memory_space=pltpu.VMEM),
        scratch_shapes=[
            pltpu.VMEM((B, D), jnp.bfloat16),
            pltpu.VMEM((B, H), jnp.bfloat16),
            pltpu.VMEM((B, D), jnp.float32),
            pltpu.VMEM((N_DEV, B, D), jnp.bfloat16),
            pltpu.VMEM((2, D, TN), jnp.float32),
            pltpu.VMEM((2, TN, D), jnp.float32),
            pltpu.SemaphoreType.DMA((2,)),
            pltpu.SemaphoreType.DMA((2,)),
            pltpu.SemaphoreType.DMA((N_LAYERS, N_DEV - 1)),
            pltpu.SemaphoreType.DMA((N_LAYERS, N_DEV - 1)),
        ],
        compiler_params=pltpu.CompilerParams(collective_id=0),
    )(x, Win0, Wout0, Win1, Wout1, Win2, Wout2)
